# Initial kernel scaffold; baseline (speedup 1.0000x reference)
import jax
import jax.numpy as jnp
from jax.experimental import pallas as pl


def kernel(x, table):
    raise NotImplementedError("write your pallas kernel here")


# speedup vs baseline: 1.5001x; 1.5001x over previous
import functools

import jax
import jax.numpy as jnp
from jax import lax
from jax.experimental import pallas as pl
from jax.experimental.pallas import tpu as pltpu
from jax.experimental.pallas import tpu_sc as plsc

_NC = 2
_NS = 16
_CH = 128


def _build(n, V, D, dtype):
    NW = _NC * _NS
    n_chunks = n // _CH
    cpt = n_chunks // NW
    G = 5
    n_groups = cpt // G
    RG = G * _CH

    mesh = plsc.VectorSubcoreMesh(
        core_axis_name="c", subcore_axis_name="s",
        num_cores=_NC, num_subcores=_NS)

    @functools.partial(
        pl.kernel,
        out_type=jax.ShapeDtypeStruct((n, D), dtype),
        mesh=mesh,
        scratch_types=[
            pltpu.VMEM((cpt, _CH), jnp.int32),
            pltpu.VMEM((2, RG, D), dtype),
            pltpu.SemaphoreType.DMA,
            pltpu.SemaphoreType.DMA,
        ],
        compiler_params=pltpu.CompilerParams(use_tc_tiling_on_sc=False),
    )
    def emb(idx_hbm, table_hbm, out_hbm, idx_v, rows, gsem0, gsem1):
        gsems = (gsem0, gsem1)
        wid = lax.axis_index("s") * _NC + lax.axis_index("c")
        chunk0 = wid * cpt
        out0 = wid * (cpt * _CH)
        pltpu.sync_copy(idx_hbm.at[pl.ds(chunk0, cpt)], idx_v)

        def fire(g, b):
            for j in range(G):
                pltpu.async_copy(
                    table_hbm.at[idx_v.at[g * G + j]],
                    rows.at[b, pl.ds(j * _CH, _CH)],
                    gsems[b],
                )

        def drain(b):
            pltpu.make_async_copy(
                table_hbm.at[pl.ds(0, RG)], rows.at[b], gsems[b]
            ).wait()

        fire(0, 0)
        fire(1, 1)

        @pl.loop(0, n_groups - 2, step=2)
        def _steady(g0):
            for b in range(2):
                g = g0 + b
                drain(b)
                pltpu.sync_copy(rows.at[b],
                                out_hbm.at[pl.ds(out0 + g * RG, RG)])
                fire(g + 2, b)

        for b in range(2):
            g = n_groups - 2 + b
            drain(b)
            pltpu.sync_copy(rows.at[b], out_hbm.at[pl.ds(out0 + g * RG, RG)])

    return emb


def kernel(x, table):
    B, H = x.shape
    V, D = table.shape
    n = B * H
    x2d = x.astype(jnp.int32).reshape(n // _CH, _CH)
    out = _build(n, V, D, table.dtype)(x2d, table)
    return out.reshape(B, H, D)
